# Initial kernel scaffold; baseline (speedup 1.0000x reference)
import jax
import jax.numpy as jnp
from jax.experimental import pallas as pl


def kernel(x, edge_index, W1, b1, W2, b2):
    raise NotImplementedError("write your pallas kernel here")


# speedup vs baseline: 10.8732x; 10.8732x over previous
import functools

import jax
import jax.numpy as jnp
from jax import lax
from jax.experimental import pallas as pl
from jax.experimental.pallas import tpu as pltpu
from jax.experimental.pallas import tpu_sc as plsc

NC = 2
NS = 16
NW = NC * NS
CH = 128


def _sc_mesh():
    return plsc.VectorSubcoreMesh(core_axis_name="c", subcore_axis_name="s")


def _sc_degree(dst_flat, n, acc_rows):
    lw = dst_flat.shape[0] // NW

    @functools.partial(
        pl.kernel,
        out_type=jax.ShapeDtypeStruct((NW * acc_rows,), jnp.float32),
        mesh=_sc_mesh(),
        scratch_types=[
            pltpu.VMEM((lw,), jnp.int32),
            pltpu.VMEM((acc_rows,), jnp.float32),
        ],
        compiler_params=pltpu.CompilerParams(needs_layout_passes=False),
    )
    def deg_kernel(dst_hbm, out_hbm, idx_v, acc_v):
        c = lax.axis_index("c")
        s = lax.axis_index("s")
        w = s * NC + c
        pltpu.sync_copy(dst_hbm.at[pl.ds(w * lw, lw)], idx_v)

        def zero(i, _):
            acc_v[pl.ds(i * 16, 16)] = jnp.zeros((16,), jnp.float32)
            return 0

        lax.fori_loop(0, acc_rows // 16, zero, 0)

        ones = jnp.ones((16,), jnp.float32)

        def body(i, _):
            v = idx_v[pl.ds(i * 16, 16)]
            plsc.addupdate_scatter(acc_v, [v], ones)
            return 0

        lax.fori_loop(0, lw // 16, body, 0)
        pltpu.sync_copy(acc_v, out_hbm.at[pl.ds(w * acc_rows, acc_rows)])

    return deg_kernel(dst_flat)


def _sc_aggregate(g, src3, dst3, acc_rows):
    n, d = g.shape
    _, K, _ = src3.shape
    assert K % 2 == 0
    rows_per_sub = (n // NS) // 8 * 8
    tail0 = NS * rows_per_sub
    tail = n - tail0

    @functools.partial(
        pl.kernel,
        out_type=jax.ShapeDtypeStruct((NC, n, d), jnp.float32),
        mesh=_sc_mesh(),
        scratch_types=[
            pltpu.VMEM((K, CH), jnp.int32),
            pltpu.VMEM((CH,), jnp.int32),
            pltpu.VMEM((CH,), jnp.int32),
            pltpu.VMEM((CH, d), jnp.float32),
            pltpu.VMEM((CH, d), jnp.float32),
            pltpu.VMEM_SHARED((acc_rows, d), jnp.float32),
            pltpu.SemaphoreType.DMA,
            pltpu.SemaphoreType.DMA,
            pltpu.SemaphoreType.DMA,
            pltpu.SemaphoreType.DMA,
        ],
    )
    def agg_kernel(g_hbm, src_hbm, dst_hbm, out_hbm,
                   didx_v, si0, si1, rows0, rows1, acc_sh,
                   semi0, semi1, semr0, semr1):
        c = lax.axis_index("c")
        s = lax.axis_index("s")
        w = s * NC + c

        stage = (si0, si1)
        rows = (rows0, rows1)
        semi = (semi0, semi1)
        semr = (semr0, semr1)

        def idx_copy(j, p):
            return pltpu.make_async_copy(src_hbm.at[w, j], stage[p], semi[p])

        def row_copy(p):
            return pltpu.make_async_copy(g_hbm.at[stage[p]], rows[p], semr[p])

        pltpu.sync_copy(dst_hbm.at[w], didx_v)
        r0 = s * rows_per_sub
        pltpu.sync_copy(g_hbm.at[pl.ds(r0, rows_per_sub)],
                        acc_sh.at[pl.ds(r0, rows_per_sub)])

        @pl.when(s == 0)
        def _():
            pltpu.sync_copy(g_hbm.at[pl.ds(tail0, tail)],
                            acc_sh.at[pl.ds(tail0, tail)])

        plsc.subcore_barrier()

        idx_copy(0, 0).start()
        idx_copy(0, 0).wait()
        row_copy(0).start()
        idx_copy(1, 1).start()

        def body(j, p):
            row_copy(p).wait()

            @pl.when(j + 2 < K)
            def _():
                idx_copy(j + 2, p).start()

            @pl.when(j + 1 < K)
            def _():
                idx_copy(j + 1, 1 - p).wait()
                row_copy(1 - p).start()

            pltpu.sync_copy(rows[p], acc_sh.at[didx_v.at[j]], add=True)

        def outer(t, u):
            body(t * 2, 0)
            body(t * 2 + 1, 1)
            return u

        lax.fori_loop(0, K // 2, outer, 0)
        plsc.subcore_barrier()
        pltpu.sync_copy(acc_sh.at[pl.ds(r0, rows_per_sub)],
                        out_hbm.at[c, pl.ds(r0, rows_per_sub)])

        @pl.when(s == 0)
        def _():
            pltpu.sync_copy(acc_sh.at[pl.ds(tail0, tail)],
                            out_hbm.at[c, pl.ds(tail0, tail)])

    return agg_kernel(g, src3, dst3)


_TC_BLK = 2000


def _tc_dinv(parts, n):
    def body(p_ref, o_ref):
        deg = jnp.sum(p_ref[...], axis=0)[:n] + 1.0
        o_ref[...] = lax.rsqrt(deg).reshape(n, 1)

    return pl.pallas_call(
        body,
        out_shape=jax.ShapeDtypeStruct((n, 1), jnp.float32),
    )(parts)


def _tc_scale_matmul(x, w, dinv):
    n, d_in = x.shape
    d_out = w.shape[1]

    def body(x_ref, w_ref, dv_ref, g_ref):
        h = jnp.dot(x_ref[...], w_ref[...], preferred_element_type=jnp.float32)
        g_ref[...] = h * dv_ref[...]

    return pl.pallas_call(
        body,
        grid=(n // _TC_BLK,),
        in_specs=[
            pl.BlockSpec((_TC_BLK, d_in), lambda i: (i, 0)),
            pl.BlockSpec((d_in, d_out), lambda i: (0, 0)),
            pl.BlockSpec((_TC_BLK, 1), lambda i: (i, 0)),
        ],
        out_specs=pl.BlockSpec((_TC_BLK, d_out), lambda i: (i, 0)),
        out_shape=jax.ShapeDtypeStruct((n, d_out), jnp.float32),
    )(x, w, dinv)


def _tc_mid(acc, g1, dinv, b1, w2):
    n, d_h = g1.shape
    d_out = w2.shape[1]

    def body(a_ref, g_ref, dv_ref, b_ref, w_ref, o_ref):
        dinv_b = dv_ref[...]
        ssum = a_ref[0] + a_ref[1] - g_ref[...]
        h = jnp.maximum(ssum * dinv_b + b_ref[...], 0.0)
        h2 = jnp.dot(h, w_ref[...], preferred_element_type=jnp.float32)
        o_ref[...] = h2 * dinv_b

    return pl.pallas_call(
        body,
        grid=(n // _TC_BLK,),
        in_specs=[
            pl.BlockSpec((2, _TC_BLK, d_h), lambda i: (0, i, 0)),
            pl.BlockSpec((_TC_BLK, d_h), lambda i: (i, 0)),
            pl.BlockSpec((_TC_BLK, 1), lambda i: (i, 0)),
            pl.BlockSpec((1, d_h), lambda i: (0, 0)),
            pl.BlockSpec((d_h, d_out), lambda i: (0, 0)),
        ],
        out_specs=pl.BlockSpec((_TC_BLK, d_out), lambda i: (i, 0)),
        out_shape=jax.ShapeDtypeStruct((n, d_out), jnp.float32),
    )(acc, g1, dinv, b1, w2)


def _tc_final(acc, g2, dinv, b2, d_out):
    n, d_pad = g2.shape

    def body(a_ref, g_ref, dv_ref, b_ref, o_ref):
        ssum = a_ref[0] + a_ref[1] - g_ref[...]
        o_ref[...] = (ssum * dv_ref[...])[:, :d_out] + b_ref[...]

    return pl.pallas_call(
        body,
        grid=(n // _TC_BLK,),
        in_specs=[
            pl.BlockSpec((2, _TC_BLK, d_pad), lambda i: (0, i, 0)),
            pl.BlockSpec((_TC_BLK, d_pad), lambda i: (i, 0)),
            pl.BlockSpec((_TC_BLK, 1), lambda i: (i, 0)),
            pl.BlockSpec((1, d_out), lambda i: (0, 0)),
        ],
        out_specs=pl.BlockSpec((_TC_BLK, d_out), lambda i: (i, 0)),
        out_shape=jax.ShapeDtypeStruct((n, d_out), jnp.float32),
    )(acc, g2, dinv, b2)


def kernel(x, edge_index, W1, b1, W2, b2):
    n = x.shape[0]
    e = edge_index.shape[1]
    epw = e // NW
    k = -(-epw // CH)
    k += k % 2
    pad = k * CH - epw
    acc_rows = -(-(n + 1) // 16) * 16

    src2 = edge_index[0].reshape(NW, epw)
    dst2 = edge_index[1].reshape(NW, epw)
    src3 = jnp.pad(src2, ((0, 0), (0, pad))).reshape(NW, k, CH)
    dst3 = jnp.pad(dst2, ((0, 0), (0, pad)),
                   constant_values=n).reshape(NW, k, CH)

    deg_flat = jnp.pad(dst2, ((0, 0), (0, pad)),
                       constant_values=n).reshape(-1)
    degp = _sc_degree(deg_flat, n, acc_rows).reshape(NW, acc_rows)
    dinv = _tc_dinv(degp, n)

    d_lat = W2.shape[1]
    w2p = jnp.zeros((W2.shape[0], 128), W2.dtype).at[:, :d_lat].set(W2)

    g1 = _tc_scale_matmul(x, W1, dinv)
    acc1 = _sc_aggregate(g1, src3, dst3, acc_rows)
    g2 = _tc_mid(acc1, g1, dinv, b1.reshape(1, -1), w2p)
    acc2 = _sc_aggregate(g2, src3, dst3, acc_rows)
    z = _tc_final(acc2, g2, dinv, b2.reshape(1, -1), d_lat)
    return z
